# baseline (speedup 1.0000x reference)
import functools

import jax
import jax.numpy as jnp
from jax import lax
from jax.experimental import pallas as pl
from jax.experimental.pallas import tpu as pltpu
from jax.experimental.pallas import tpu_sc as plsc

_E = 10
_K = 3
_EP = 16
_BT = 1024


def _gating_body(x_ref, w_ref, b_ref, nb_ref, logits_ref, weights_ref, wpad_ref):
    y = jnp.dot(x_ref[...], w_ref[...], preferred_element_type=jnp.float32)
    y = y + b_ref[...]
    noise_std = jax.nn.softplus(y[:, _EP:])
    logits = y[:, :_EP] + nb_ref[...] * noise_std
    logits_ref[...] = logits[:, :_E]
    m = jnp.max(logits, axis=1, keepdims=True)
    e = jnp.exp(logits - m)
    w = e / jnp.sum(e, axis=1, keepdims=True)
    weights_ref[...] = w[:, :_E]
    wpad_ref[...] = w


def _gating(x, nb, Wc, bc):
    T, D = x.shape
    grid = (T // _BT,)
    return pl.pallas_call(
        _gating_body,
        grid=grid,
        in_specs=[
            pl.BlockSpec((_BT, D), lambda i: (i, 0)),
            pl.BlockSpec((D, 2 * _EP), lambda i: (0, 0)),
            pl.BlockSpec((1, 2 * _EP), lambda i: (0, 0)),
            pl.BlockSpec((_BT, _EP), lambda i: (i, 0)),
        ],
        out_specs=[
            pl.BlockSpec((_BT, _E), lambda i: (i, 0)),
            pl.BlockSpec((_BT, _E), lambda i: (i, 0)),
            pl.BlockSpec((_BT, _EP), lambda i: (i, 0)),
        ],
        out_shape=[
            jax.ShapeDtypeStruct((T, _E), jnp.float32),
            jax.ShapeDtypeStruct((T, _E), jnp.float32),
            jax.ShapeDtypeStruct((T, _EP), jnp.float32),
        ],
        compiler_params=pltpu.CompilerParams(
            dimension_semantics=("arbitrary",),
        ),
    )(x, Wc, bc, nb)


def _make_sc_router(T, half, hoff, nc, ns):
    nw = nc * ns
    tpw = half // nw
    mesh = plsc.VectorSubcoreMesh(core_axis_name="c", subcore_axis_name="s")

    @functools.partial(
        pl.kernel,
        out_type=jax.ShapeDtypeStruct((half, _EP), jnp.float32),
        mesh=mesh,
        scratch_types=[
            pltpu.VMEM((tpw * _EP,), jnp.float32),
            pltpu.VMEM((tpw, _EP), jnp.float32),
            pltpu.VMEM((_EP,), jnp.int32),
        ],
        compiler_params=pltpu.CompilerParams(needs_layout_passes=False),
    )
    def sc_router(wpad_hbm, keep_hbm, p_hbm, in_v, p_v, k_v):
        wid = lax.axis_index("s") * nc + lax.axis_index("c")
        base = hoff * _EP + wid * tpw * _EP
        pltpu.sync_copy(wpad_hbm.at[pl.ds(base, tpw * _EP)], in_v)
        pltpu.sync_copy(keep_hbm, k_v)
        lanes = lax.iota(jnp.int32, _EP)
        keepb = k_v[...] != 0
        emask = lanes < _E
        p1 = lanes ^ 1
        p2 = lanes ^ 2
        pidx = (lanes + _EP - _E) & (_EP - 1)
        zeros = jnp.zeros((_EP,), jnp.float32)

        @plsc.parallel_loop(0, tpw, unroll=4)
        def _route(t):
            w = in_v[pl.ds(t * _EP, _EP)]
            wk = jnp.where(emask, w, -1.0)
            sk, sv = plsc.sort_key_val(wk, lanes, descending=True)
            kept = jnp.where(keepb, sk, 0.0)
            s = kept + jnp.take(kept, p1)
            s = s + jnp.take(s, p2)
            norm = kept / s
            svf = plsc.bitcast(jnp.take(sv, pidx), jnp.float32)
            base_row = jnp.where(emask, zeros,
                                 jnp.where(lanes < _E + _K, svf, zeros))
            p_v[t] = base_row
            trow = jnp.full((_EP,), t, jnp.int32)
            plsc.store_scatter(p_v, [trow, sv], norm, mask=keepb)

        pltpu.sync_copy(p_v, p_hbm.at[pl.ds(wid * tpw, tpw), :])

    return sc_router


def kernel(x, noise_base, Wg, bg, Wn, bn, top_k):
    T, D = x.shape
    E = Wg.shape[0]
    Wc = (jnp.zeros((D, 2 * _EP), jnp.float32)
          .at[:, :E].set(Wg.T).at[:, _EP:_EP + E].set(Wn.T))
    bc = (jnp.full((1, 2 * _EP), 0.0, jnp.float32)
          .at[0, :E].set(bg)
          .at[0, E:_EP].set(-1e30)
          .at[0, _EP:_EP + E].set(bn))
    nb = jnp.pad(noise_base, ((0, 0), (0, _EP - E)))
    keep = (jnp.arange(_EP, dtype=jnp.int32)
            < jnp.minimum(jnp.asarray(top_k, jnp.int32), _K)).astype(jnp.int32)

    logits, weights, wpad = _gating(x, nb, Wc, bc)

    info = plsc.get_sparse_core_info()
    half = T // 2
    wpad_flat = wpad.reshape(-1)
    halves = []
    for h in range(2):
        router = _make_sc_router(T, half, h * half,
                                 info.num_cores, info.num_subcores)
        halves.append(router(wpad_flat, keep))

    pfull = jnp.concatenate(halves, axis=0)
    dense = pfull[:, :_E]
    idx = lax.bitcast_convert_type(pfull[:, _E:_E + _K], jnp.int32)
    return (dense, weights, logits, idx)


# speedup vs baseline: 1.0445x; 1.0445x over previous
import functools

import jax
import jax.numpy as jnp
from jax import lax
from jax.experimental import pallas as pl
from jax.experimental.pallas import tpu as pltpu
from jax.experimental.pallas import tpu_sc as plsc

_E = 10
_K = 3
_EP = 16
_BT = 2048


def _gating_body(x_ref, w_ref, b_ref, nb_ref, logits_ref, weights_ref, wpad_ref):
    y = jnp.dot(x_ref[...], w_ref[...], preferred_element_type=jnp.float32)
    y = y + b_ref[...]
    noise_std = jax.nn.softplus(y[:, _EP:])
    logits = y[:, :_EP] + nb_ref[...] * noise_std
    logits_ref[...] = logits[:, :_E]
    m = jnp.max(logits, axis=1, keepdims=True)
    e = jnp.exp(logits - m)
    w = e / jnp.sum(e, axis=1, keepdims=True)
    weights_ref[...] = w[:, :_E]
    wpad_ref[...] = w


def _gating(x, nb, Wc, bc):
    T, D = x.shape
    grid = (T // _BT,)
    return pl.pallas_call(
        _gating_body,
        grid=grid,
        in_specs=[
            pl.BlockSpec((_BT, D), lambda i: (i, 0)),
            pl.BlockSpec((D, 2 * _EP), lambda i: (0, 0)),
            pl.BlockSpec((1, 2 * _EP), lambda i: (0, 0)),
            pl.BlockSpec((_BT, _EP), lambda i: (i, 0)),
        ],
        out_specs=[
            pl.BlockSpec((_BT, _E), lambda i: (i, 0)),
            pl.BlockSpec((_BT, _E), lambda i: (i, 0)),
            pl.BlockSpec((_BT, _EP), lambda i: (i, 0)),
        ],
        out_shape=[
            jax.ShapeDtypeStruct((T, _E), jnp.float32),
            jax.ShapeDtypeStruct((T, _E), jnp.float32),
            jax.ShapeDtypeStruct((T, _EP), jnp.float32),
        ],
        compiler_params=pltpu.CompilerParams(
            dimension_semantics=("arbitrary",),
        ),
    )(x, Wc, bc, nb)


def _make_sc_router(T, half, hoff, nc, ns):
    nw = nc * ns
    tpw = half // nw
    mesh = plsc.VectorSubcoreMesh(core_axis_name="c", subcore_axis_name="s")

    @functools.partial(
        pl.kernel,
        out_type=jax.ShapeDtypeStruct((half, _EP), jnp.float32),
        mesh=mesh,
        scratch_types=[
            pltpu.VMEM((tpw * _EP,), jnp.float32),
            pltpu.VMEM((tpw, _EP), jnp.float32),
            pltpu.VMEM((_EP,), jnp.int32),
        ],
        compiler_params=pltpu.CompilerParams(needs_layout_passes=False),
    )
    def sc_router(wpad_hbm, keep_hbm, p_hbm, in_v, p_v, k_v):
        wid = lax.axis_index("s") * nc + lax.axis_index("c")
        base = hoff * _EP + wid * tpw * _EP
        pltpu.sync_copy(wpad_hbm.at[pl.ds(base, tpw * _EP)], in_v)
        pltpu.sync_copy(keep_hbm, k_v)
        lanes = lax.iota(jnp.int32, _EP)
        keepb = k_v[...] != 0
        emask = lanes < _E
        p1 = lanes ^ 1
        p2 = lanes ^ 2
        pidx = (lanes + _EP - _E) & (_EP - 1)
        zeros = jnp.zeros((_EP,), jnp.float32)

        @plsc.parallel_loop(0, tpw, unroll=4)
        def _route(t):
            w = in_v[pl.ds(t * _EP, _EP)]
            wk = jnp.where(emask, w, -1.0)
            sk, sv = plsc.sort_key_val(wk, lanes, descending=True)
            kept = jnp.where(keepb, sk, 0.0)
            s = kept + jnp.take(kept, p1)
            s = s + jnp.take(s, p2)
            norm = kept / s
            svf = plsc.bitcast(jnp.take(sv, pidx), jnp.float32)
            base_row = jnp.where(emask, zeros,
                                 jnp.where(lanes < _E + _K, svf, zeros))
            p_v[t] = base_row
            trow = jnp.full((_EP,), t, jnp.int32)
            plsc.store_scatter(p_v, [trow, sv], norm, mask=keepb)

        pltpu.sync_copy(p_v, p_hbm.at[pl.ds(wid * tpw, tpw), :])

    return sc_router


def kernel(x, noise_base, Wg, bg, Wn, bn, top_k):
    T, D = x.shape
    E = Wg.shape[0]
    Wc = (jnp.zeros((D, 2 * _EP), jnp.float32)
          .at[:, :E].set(Wg.T).at[:, _EP:_EP + E].set(Wn.T))
    bc = (jnp.full((1, 2 * _EP), 0.0, jnp.float32)
          .at[0, :E].set(bg)
          .at[0, E:_EP].set(-1e30)
          .at[0, _EP:_EP + E].set(bn))
    nb = jnp.pad(noise_base, ((0, 0), (0, _EP - E)))
    keep = (jnp.arange(_EP, dtype=jnp.int32)
            < jnp.minimum(jnp.asarray(top_k, jnp.int32), _K)).astype(jnp.int32)

    logits, weights, wpad = _gating(x, nb, Wc, bc)

    info = plsc.get_sparse_core_info()
    half = T // 2
    wpad_flat = wpad.reshape(-1)
    halves = []
    for h in range(2):
        router = _make_sc_router(T, half, h * half,
                                 info.num_cores, info.num_subcores)
        halves.append(router(wpad_flat, keep))

    pfull = jnp.concatenate(halves, axis=0)
    dense = pfull[:, :_E]
    idx = lax.bitcast_convert_type(pfull[:, _E:_E + _K], jnp.int32)
    return (dense, weights, logits, idx)
